# baseline (speedup 1.0000x reference)
import functools

import jax
import jax.numpy as jnp
from jax import lax
from jax.experimental import pallas as pl
from jax.experimental.pallas import tpu as pltpu
from jax.experimental.pallas import tpu_sc as plsc

EMB_DIM = 128
NUM_EDGES = 320000
NC = 2
NS = 16
NW = NC * NS
BPW = NUM_EDGES // NW
CH = 80
NCH = BPW // CH
NB = 5
N_OUTER = NCH // NB


@functools.cache
def _build_gather_kernel():
    @functools.partial(
        pl.kernel,
        mesh=plsc.VectorSubcoreMesh(core_axis_name="c", subcore_axis_name="s"),
        out_type=jax.ShapeDtypeStruct((NUM_EDGES, EMB_DIM), jnp.float32),
        scratch_types=(
            [pltpu.VMEM((NCH, CH), jnp.int32),
             pltpu.VMEM((NB, CH, EMB_DIM), jnp.float32)]
            + [pltpu.SemaphoreType.DMA] * NB
        ),
    )
    def _gather_kernel(idx_hbm, table_hbm, out_hbm, idx_v, rows_v, *gsems):
        cid = lax.axis_index("c")
        sid = lax.axis_index("s")
        wid = sid * NC + cid
        row0 = wid * BPW
        pltpu.sync_copy(idx_hbm.at[wid], idx_v)

        def body(g, carry):
            jbase = g * NB
            descs = []
            for b in range(NB):
                descs.append(pltpu.async_copy(
                    table_hbm.at[idx_v.at[jbase + b]], rows_v.at[b], gsems[b]))
            for b in range(NB):
                descs[b].wait()
                pltpu.sync_copy(
                    rows_v.at[b],
                    out_hbm.at[pl.ds(row0 + (jbase + b) * CH, CH)])
            return carry

        lax.fori_loop(0, N_OUTER, body, 0)

    return _gather_kernel


def kernel(edge_attr, emb_table_0):
    idx = edge_attr.reshape(NW, NCH, CH).astype(jnp.int32)
    return _build_gather_kernel()(idx, emb_table_0)
